# Initial kernel scaffold; baseline (speedup 1.0000x reference)
import jax
import jax.numpy as jnp
from jax.experimental import pallas as pl


def kernel(x, edge_index, adj_vals, W, b):
    raise NotImplementedError("write your pallas kernel here")


# speedup vs baseline: 5.4423x; 5.4423x over previous
import functools

import jax
import jax.numpy as jnp
from jax import lax
from jax.experimental import pallas as pl
from jax.experimental.pallas import tpu as pltpu
from jax.experimental.pallas import tpu_sc as plsc

N = 10000
E = 320000
D = 128
L = 16
NC = 2
NS = 16
NW = NC * NS
NP = NS * 640
RPT = NP // NS
EPW = E // NW
C = 80
NCHUNK = EPW // C
KSUP = 25
NSUP = NCHUNK // KSUP
GROUPS = C // L


def _sc_body(x_hbm, src_hbm, dst_hbm, vals_hbm, p_hbm, deg_hbm,
             src_v, dst_v, vals_v, rows_v, zdeg_v, acc_sh, dacc_sh, sem):
    cid = lax.axis_index("c")
    sid = lax.axis_index("s")
    wid = sid * NC + cid

    zeros16 = jnp.zeros((L,), jnp.float32)

    def zrow(r, carry):
        for j in range(D // L):
            rows_v[r, pl.ds(j * L, L)] = zeros16
        return carry

    lax.fori_loop(0, C, zrow, 0)
    for j in range(RPT // L):
        zdeg_v[pl.ds(j * L, L)] = zeros16

    row0 = pl.multiple_of(sid * RPT, 8)
    for k in range(RPT // C):
        pltpu.sync_copy(rows_v, acc_sh.at[pl.ds(row0 + k * C, C)])
    pltpu.sync_copy(zdeg_v, dacc_sh.at[pl.ds(row0, RPT)])
    plsc.subcore_barrier()

    def super_body(sbi, carry):
        pltpu.sync_copy(src_hbm.at[wid, sbi], src_v)
        pltpu.sync_copy(dst_hbm.at[wid, sbi], dst_v)
        pltpu.sync_copy(vals_hbm.at[wid, sbi], vals_v)

        def chunk_body(i, carry1):
            pltpu.async_copy(x_hbm.at[src_v.at[i]], rows_v, sem).wait()

            def group_body(g, carry2):
                vv = vals_v[i, pl.ds(g * L, L)]
                for r in range(L):
                    s = vv.at[jnp.full((L,), r, jnp.int32)].get(
                        mode="promise_in_bounds")
                    row = g * L + r
                    for j in range(D // L):
                        sl = pl.ds(j * L, L)
                        rows_v[row, sl] = rows_v[row, sl] * s
                return carry2

            lax.fori_loop(0, GROUPS, group_body, 0)

            pltpu.sync_copy(rows_v, acc_sh.at[dst_v.at[i]], add=True)
            pltpu.sync_copy(vals_v.at[i], dacc_sh.at[dst_v.at[i]], add=True)
            return carry1

        lax.fori_loop(0, KSUP, chunk_body, 0)
        return carry

    lax.fori_loop(0, NSUP, super_body, 0)
    plsc.subcore_barrier()

    pltpu.sync_copy(acc_sh.at[pl.ds(row0, RPT)], p_hbm.at[cid, pl.ds(row0, RPT)])
    pltpu.sync_copy(dacc_sh.at[pl.ds(row0, RPT)], deg_hbm.at[cid, pl.ds(row0, RPT)])


_sc_scatter = functools.partial(
    pl.kernel,
    out_type=[
        jax.ShapeDtypeStruct((NC, NP, D), jnp.float32),
        jax.ShapeDtypeStruct((NC, NP), jnp.float32),
    ],
    mesh=plsc.VectorSubcoreMesh(core_axis_name="c", subcore_axis_name="s"),
    scratch_types=[
        pltpu.VMEM((KSUP, C), jnp.int32),
        pltpu.VMEM((KSUP, C), jnp.int32),
        pltpu.VMEM((KSUP, C), jnp.float32),
        pltpu.VMEM((C, D), jnp.float32),
        pltpu.VMEM((RPT,), jnp.float32),
        pltpu.VMEM_SHARED((NP, D), jnp.float32),
        pltpu.VMEM_SHARED((NP,), jnp.float32),
        pltpu.SemaphoreType.DMA,
    ],
)(_sc_body)


def _mm_body(p0_ref, p1_ref, d0_ref, d1_ref, wt_ref, b_ref, o_ref):
    h = p0_ref[...] + p1_ref[...]
    dd = d0_ref[...] + d1_ref[...]
    o_ref[...] = (jnp.dot(h, wt_ref[...], preferred_element_type=jnp.float32)
                  + dd * b_ref[...])


_R = 512


def _tc_matmul(p0, p1, d0, d1, wt, b2):
    return pl.pallas_call(
        _mm_body,
        grid=(NP // _R,),
        in_specs=[
            pl.BlockSpec((_R, D), lambda i: (i, 0)),
            pl.BlockSpec((_R, D), lambda i: (i, 0)),
            pl.BlockSpec((_R, 1), lambda i: (i, 0)),
            pl.BlockSpec((_R, 1), lambda i: (i, 0)),
            pl.BlockSpec((D, D), lambda i: (0, 0)),
            pl.BlockSpec((1, D), lambda i: (0, 0)),
        ],
        out_specs=pl.BlockSpec((_R, D), lambda i: (i, 0)),
        out_shape=jax.ShapeDtypeStruct((NP, D), jnp.float32),
    )(p0, p1, d0, d1, wt, b2)


def kernel(x, edge_index, adj_vals, W, b):
    src = edge_index[1].astype(jnp.int32).reshape(NW, NSUP, KSUP, C)
    dst = edge_index[0].astype(jnp.int32).reshape(NW, NSUP, KSUP, C)
    vals = adj_vals.reshape(NW, NSUP, KSUP, C)
    P, deg = _sc_scatter(x, src, dst, vals)
    out = _tc_matmul(P[0], P[1], deg[0][:, None], deg[1][:, None],
                     W.T, b[None, :])
    return out[:N]


# speedup vs baseline: 8.2735x; 1.5202x over previous
import functools

import jax
import jax.numpy as jnp
from jax import lax
from jax.experimental import pallas as pl
from jax.experimental.pallas import tpu as pltpu
from jax.experimental.pallas import tpu_sc as plsc

N = 10000
E = 320000
D = 128
L = 16
NC = 2
NS = 16
NW = NC * NS
NP = NS * 640
RPT = NP // NS
EPW = E // NW
C = 80
NCHUNK = EPW // C
KSUP = 25
NSUP = NCHUNK // KSUP
GROUPS = C // L


def _scale_rows(rows_v, vals_v, i):
    def group_body(g, carry2):
        vv = vals_v[i, pl.ds(g * L, L)]
        for r in range(L):
            s = vv.at[jnp.full((L,), r, jnp.int32)].get(
                mode="promise_in_bounds")
            row = g * L + r
            for j in range(D // L):
                sl = pl.ds(j * L, L)
                rows_v[row, sl] = rows_v[row, sl] * s
        return carry2

    lax.fori_loop(0, GROUPS, group_body, 0)


def _sc_body(x_hbm, src_hbm, dst_hbm, vals_hbm, p_hbm, deg_hbm,
             src_v, dst_v, vals_v, rows0_v, rows1_v, zdeg_v,
             acc_sh, dacc_sh, gsem0, gsem1, ssem0, ssem1, dsem):
    cid = lax.axis_index("c")
    sid = lax.axis_index("s")
    wid = sid * NC + cid

    zeros16 = jnp.zeros((L,), jnp.float32)

    def zrow(r, carry):
        for j in range(D // L):
            rows0_v[r, pl.ds(j * L, L)] = zeros16
        return carry

    lax.fori_loop(0, C, zrow, 0)
    for j in range(RPT // L):
        zdeg_v[pl.ds(j * L, L)] = zeros16

    row0 = pl.multiple_of(sid * RPT, 8)
    for k in range(RPT // C):
        pltpu.sync_copy(rows0_v, acc_sh.at[pl.ds(row0 + k * C, C)])
    pltpu.sync_copy(zdeg_v, dacc_sh.at[pl.ds(row0, RPT)])
    plsc.subcore_barrier()

    def step(i, cur_rows, cur_gsem, cur_ssem, nxt_rows, nxt_gsem, nxt_ssem):
        @pl.when(i > 0)
        def _():
            pltpu.make_async_copy(
                nxt_rows, acc_sh.at[pl.ds(0, C)], nxt_ssem).wait()

        @pl.when(i < KSUP - 1)
        def _():
            pltpu.async_copy(x_hbm.at[src_v.at[i + 1]], nxt_rows, nxt_gsem)

        pltpu.make_async_copy(
            x_hbm.at[src_v.at[i]], cur_rows, cur_gsem).wait()

        _scale_rows(cur_rows, vals_v, i)

        pltpu.async_copy(cur_rows, acc_sh.at[dst_v.at[i]], cur_ssem, add=True)
        pltpu.async_copy(vals_v.at[i], dacc_sh.at[dst_v.at[i]], dsem, add=True)

    def super_body(sbi, carry):
        pltpu.sync_copy(src_hbm.at[wid, sbi], src_v)
        pltpu.sync_copy(dst_hbm.at[wid, sbi], dst_v)
        pltpu.sync_copy(vals_hbm.at[wid, sbi], vals_v)

        pltpu.async_copy(x_hbm.at[src_v.at[0]], rows0_v, gsem0)

        def chunk_iter(i, carry1):
            @pl.when(i % 2 == 0)
            def _():
                step(i, rows0_v, gsem0, ssem0, rows1_v, gsem1, ssem1)

            @pl.when(i % 2 == 1)
            def _():
                step(i, rows1_v, gsem1, ssem1, rows0_v, gsem0, ssem0)

            return carry1

        lax.fori_loop(0, KSUP, chunk_iter, 0)

        pltpu.make_async_copy(
            rows0_v if (KSUP - 1) % 2 == 0 else rows1_v,
            acc_sh.at[pl.ds(0, C)],
            ssem0 if (KSUP - 1) % 2 == 0 else ssem1).wait()

        def drain_deg(i, carry2):
            pltpu.make_async_copy(
                vals_v.at[0], dacc_sh.at[dst_v.at[0]], dsem).wait()
            return carry2

        lax.fori_loop(0, KSUP, drain_deg, 0)
        return carry

    lax.fori_loop(0, NSUP, super_body, 0)
    plsc.subcore_barrier()

    pltpu.sync_copy(acc_sh.at[pl.ds(row0, RPT)], p_hbm.at[cid, pl.ds(row0, RPT)])
    pltpu.sync_copy(dacc_sh.at[pl.ds(row0, RPT)], deg_hbm.at[cid, pl.ds(row0, RPT)])


_sc_scatter = functools.partial(
    pl.kernel,
    out_type=[
        jax.ShapeDtypeStruct((NC, NP, D), jnp.float32),
        jax.ShapeDtypeStruct((NC, NP), jnp.float32),
    ],
    mesh=plsc.VectorSubcoreMesh(core_axis_name="c", subcore_axis_name="s"),
    scratch_types=[
        pltpu.VMEM((KSUP, C), jnp.int32),
        pltpu.VMEM((KSUP, C), jnp.int32),
        pltpu.VMEM((KSUP, C), jnp.float32),
        pltpu.VMEM((C, D), jnp.float32),
        pltpu.VMEM((C, D), jnp.float32),
        pltpu.VMEM((RPT,), jnp.float32),
        pltpu.VMEM_SHARED((NP, D), jnp.float32),
        pltpu.VMEM_SHARED((NP,), jnp.float32),
        pltpu.SemaphoreType.DMA,
        pltpu.SemaphoreType.DMA,
        pltpu.SemaphoreType.DMA,
        pltpu.SemaphoreType.DMA,
        pltpu.SemaphoreType.DMA,
    ],
)(_sc_body)


def _mm_body(p0_ref, p1_ref, d0_ref, d1_ref, wt_ref, b_ref, o_ref):
    h = p0_ref[...] + p1_ref[...]
    dd = d0_ref[...] + d1_ref[...]
    o_ref[...] = (jnp.dot(h, wt_ref[...], preferred_element_type=jnp.float32)
                  + dd * b_ref[...])


_R = 512


def _tc_matmul(p0, p1, d0, d1, wt, b2):
    return pl.pallas_call(
        _mm_body,
        grid=(NP // _R,),
        in_specs=[
            pl.BlockSpec((_R, D), lambda i: (i, 0)),
            pl.BlockSpec((_R, D), lambda i: (i, 0)),
            pl.BlockSpec((_R, 1), lambda i: (i, 0)),
            pl.BlockSpec((_R, 1), lambda i: (i, 0)),
            pl.BlockSpec((D, D), lambda i: (0, 0)),
            pl.BlockSpec((1, D), lambda i: (0, 0)),
        ],
        out_specs=pl.BlockSpec((_R, D), lambda i: (i, 0)),
        out_shape=jax.ShapeDtypeStruct((NP, D), jnp.float32),
    )(p0, p1, d0, d1, wt, b2)


def kernel(x, edge_index, adj_vals, W, b):
    src = edge_index[1].astype(jnp.int32).reshape(NW, NSUP, KSUP, C)
    dst = edge_index[0].astype(jnp.int32).reshape(NW, NSUP, KSUP, C)
    vals = adj_vals.reshape(NW, NSUP, KSUP, C)
    P, deg = _sc_scatter(x, src, dst, vals)
    out = _tc_matmul(P[0], P[1], deg[0][:, None], deg[1][:, None],
                     W.T, b[None, :])
    return out[:N]


# speedup vs baseline: 8.5375x; 1.0319x over previous
import functools

import jax
import jax.numpy as jnp
from jax import lax
from jax.experimental import pallas as pl
from jax.experimental.pallas import tpu as pltpu
from jax.experimental.pallas import tpu_sc as plsc

N = 10000
E = 320000
D = 128
L = 16
NC = 2
NS = 16
NW = NC * NS
NP = NS * 640
RPT = NP // NS
EPW = E // NW
C = 80
NCHUNK = EPW // C
KSUP = 25
NSUP = NCHUNK // KSUP
GROUPS = C // L


def _scale_rows(rows_v, vals_v, i):
    def group_body(g, carry2):
        vv = vals_v[i, pl.ds(g * L, L)]
        for r in range(L):
            s = vv.at[jnp.full((L,), r, jnp.int32)].get(
                mode="promise_in_bounds")
            row = g * L + r
            for j in range(D // L):
                sl = pl.ds(j * L, L)
                rows_v[row, sl] = rows_v[row, sl] * s
        return carry2

    lax.fori_loop(0, GROUPS, group_body, 0)


def _sc_body(x_hbm, src_hbm, dst_hbm, vals_hbm, p_hbm, deg_hbm,
             src_v, dst_v, vals_v, rows0_v, rows1_v, zdeg_v,
             acc_sh, dacc_sh, gsem0, gsem1, ssem0, ssem1, dsem):
    cid = lax.axis_index("c")
    sid = lax.axis_index("s")
    wid = sid * NC + cid

    zeros16 = jnp.zeros((L,), jnp.float32)

    def zrow(r, carry):
        for j in range(D // L):
            rows0_v[r, pl.ds(j * L, L)] = zeros16
        return carry

    lax.fori_loop(0, C, zrow, 0)
    for j in range(RPT // L):
        zdeg_v[pl.ds(j * L, L)] = zeros16

    row0 = pl.multiple_of(sid * RPT, 8)
    for k in range(RPT // C):
        pltpu.sync_copy(rows0_v, acc_sh.at[pl.ds(row0 + k * C, C)])
    pltpu.sync_copy(zdeg_v, dacc_sh.at[pl.ds(row0, RPT)])
    plsc.subcore_barrier()

    def step(i, cur_rows, cur_gsem, cur_ssem, nxt_rows, nxt_gsem, nxt_ssem):
        @pl.when(i > 0)
        def _():
            pltpu.make_async_copy(
                nxt_rows, acc_sh.at[pl.ds(0, C)], nxt_ssem).wait()

        @pl.when(i < KSUP - 1)
        def _():
            pltpu.async_copy(x_hbm.at[src_v.at[i + 1]], nxt_rows, nxt_gsem)

        pltpu.make_async_copy(
            x_hbm.at[src_v.at[i]], cur_rows, cur_gsem).wait()

        _scale_rows(cur_rows, vals_v, i)

        pltpu.async_copy(cur_rows, acc_sh.at[dst_v.at[i]], cur_ssem, add=True)
        pltpu.async_copy(vals_v.at[i], dacc_sh.at[dst_v.at[i]], dsem, add=True)

    def super_body(sbi, carry):
        pltpu.sync_copy(src_hbm.at[wid, sbi], src_v)
        pltpu.sync_copy(dst_hbm.at[wid, sbi], dst_v)
        pltpu.sync_copy(vals_hbm.at[wid, sbi], vals_v)

        pltpu.async_copy(x_hbm.at[src_v.at[0]], rows0_v, gsem0)

        def chunk_iter(i, carry1):
            @pl.when(i % 2 == 0)
            def _():
                step(i, rows0_v, gsem0, ssem0, rows1_v, gsem1, ssem1)

            @pl.when(i % 2 == 1)
            def _():
                step(i, rows1_v, gsem1, ssem1, rows0_v, gsem0, ssem0)

            return carry1

        lax.fori_loop(0, KSUP, chunk_iter, 0)

        pltpu.make_async_copy(
            rows0_v if (KSUP - 1) % 2 == 0 else rows1_v,
            acc_sh.at[pl.ds(0, C)],
            ssem0 if (KSUP - 1) % 2 == 0 else ssem1).wait()

        def drain_deg(i, carry2):
            pltpu.make_async_copy(
                vals_v.at[0], dacc_sh.at[dst_v.at[0]], dsem).wait()
            return carry2

        lax.fori_loop(0, KSUP, drain_deg, 0)
        return carry

    lax.fori_loop(0, NSUP, super_body, 0)
    plsc.subcore_barrier()

    pltpu.sync_copy(acc_sh.at[pl.ds(row0, RPT)], p_hbm.at[cid, pl.ds(row0, RPT)])
    pltpu.sync_copy(dacc_sh.at[pl.ds(row0, RPT)], deg_hbm.at[cid, pl.ds(row0, RPT)])


_sc_scatter = functools.partial(
    pl.kernel,
    out_type=[
        jax.ShapeDtypeStruct((NC, NP, D), jnp.float32),
        jax.ShapeDtypeStruct((NC, NP), jnp.float32),
    ],
    mesh=plsc.VectorSubcoreMesh(core_axis_name="c", subcore_axis_name="s"),
    scratch_types=[
        pltpu.VMEM((KSUP, C), jnp.int32),
        pltpu.VMEM((KSUP, C), jnp.int32),
        pltpu.VMEM((KSUP, C), jnp.float32),
        pltpu.VMEM((C, D), jnp.float32),
        pltpu.VMEM((C, D), jnp.float32),
        pltpu.VMEM((RPT,), jnp.float32),
        pltpu.VMEM_SHARED((NP, D), jnp.float32),
        pltpu.VMEM_SHARED((NP,), jnp.float32),
        pltpu.SemaphoreType.DMA,
        pltpu.SemaphoreType.DMA,
        pltpu.SemaphoreType.DMA,
        pltpu.SemaphoreType.DMA,
        pltpu.SemaphoreType.DMA,
    ],
)(_sc_body)


def _mm_body(p0_ref, p1_ref, d0_ref, d1_ref, wt_ref, b_ref, o_ref):
    h = p0_ref[...] + p1_ref[...]
    dd = d0_ref[...] + d1_ref[...]
    o_ref[...] = (jnp.dot(h, wt_ref[...], preferred_element_type=jnp.float32)
                  + dd * b_ref[...])


_R = 2048


def _tc_matmul(p0, p1, d0, d1, wt, b2):
    return pl.pallas_call(
        _mm_body,
        grid=(NP // _R,),
        in_specs=[
            pl.BlockSpec((_R, D), lambda i: (i, 0)),
            pl.BlockSpec((_R, D), lambda i: (i, 0)),
            pl.BlockSpec((_R, 1), lambda i: (i, 0)),
            pl.BlockSpec((_R, 1), lambda i: (i, 0)),
            pl.BlockSpec((D, D), lambda i: (0, 0)),
            pl.BlockSpec((1, D), lambda i: (0, 0)),
        ],
        out_specs=pl.BlockSpec((_R, D), lambda i: (i, 0)),
        out_shape=jax.ShapeDtypeStruct((NP, D), jnp.float32),
    )(p0, p1, d0, d1, wt, b2)


def kernel(x, edge_index, adj_vals, W, b):
    src = edge_index[1].astype(jnp.int32).reshape(NW, NSUP, KSUP, C)
    dst = edge_index[0].astype(jnp.int32).reshape(NW, NSUP, KSUP, C)
    vals = adj_vals.reshape(NW, NSUP, KSUP, C)
    P, deg = _sc_scatter(x, src, dst, vals)
    out = _tc_matmul(P[0], P[1], deg[0][:, None], deg[1][:, None],
                     W.T, b[None, :])
    return out[:N]
